# baseline (speedup 1.0000x reference)
import jax
import jax.numpy as jnp
from jax import lax
from jax.experimental import pallas as pl
from jax.experimental.pallas import tpu as pltpu

_PROGRESS = min(1.0 / 1000.0, 1.0)
_TAU = 2.0 - _PROGRESS * (2.0 - 0.5)

_ROWS = 4096
_IN_DIM = 4096
_STATE_DIM = 2048
_E = 64
_KSEL = 8
_BLK = 256
_NBLK = _ROWS // _BLK
_N_BISECT = 30

_INTERPRET = False


def _keys_body(a_ref, wk_ref, k_ref):
    a = a_ref[...].astype(jnp.bfloat16)
    wk = wk_ref[...].astype(jnp.bfloat16)
    k_ref[...] = lax.dot_general(a, wk, (((1,), (1,)), ((), ())),
                                 preferred_element_type=jnp.float32)


def _entmax_topk(s, idx_ref, w_ref):
    x = s - jnp.max(s, axis=-1, keepdims=True)
    l = jnp.full((_BLK, 1), -2.0, dtype=jnp.float32)
    r = jnp.zeros((_BLK, 1), dtype=jnp.float32)
    for _ in range(_N_BISECT):
        mid = (l + r) * 0.5
        y = jnp.maximum(x - mid, 0.0)
        vm = jnp.sum(y * y, axis=-1, keepdims=True) - 1.0
        gt = vm > 0.0
        l = jnp.where(gt, mid, l)
        r = jnp.where(gt, r, mid)
    tau_b = (l + r) * 0.5
    yy = jnp.maximum(x - tau_b, 0.0)
    sup = yy * yy
    p = sup / (jnp.sum(sup, axis=-1, keepdims=True) + 1e-12)
    iota = lax.broadcasted_iota(jnp.int32, (_BLK, _E), 1)
    vals = []
    idxs = []
    pw = p
    for _ in range(_KSEL):
        m = jnp.max(pw, axis=-1, keepdims=True)
        cand = jnp.where(pw == m, iota, _E)
        am = jnp.min(cand, axis=-1, keepdims=True)
        vals.append(m)
        idxs.append(am)
        pw = jnp.where(iota == am, -1.0, pw)
    v = jnp.concatenate(vals, axis=1)
    w_ref[...] = v / (jnp.sum(v, axis=-1, keepdims=True) + 1e-12)
    idx_ref[...] = jnp.concatenate(idxs, axis=1)


def _router_body(h_ref, wq_ref, k_ref, idx_ref, w_ref, s_scratch):
    i = pl.program_id(0)

    @pl.when(i < _NBLK)
    def _matmul():
        h = h_ref[...].astype(jnp.bfloat16)
        q = lax.dot_general(h, wq_ref[...], (((1,), (1,)), ((), ())),
                            preferred_element_type=jnp.float32)
        qb = q.astype(jnp.bfloat16)
        kb = k_ref[...].astype(jnp.bfloat16)
        s = lax.dot_general(qb, kb, (((1,), (1,)), ((), ())),
                            preferred_element_type=jnp.float32) / _TAU
        s_scratch[lax.rem(i, 2)] = s

    @pl.when(i > 0)
    def _vector():
        _entmax_topk(s_scratch[lax.rem(i + 1, 2)], idx_ref, w_ref)


def kernel(h_concat, A_states, W_Q, W_K):
    wq_bf = W_Q.astype(jnp.bfloat16)
    k_mat = pl.pallas_call(
        _keys_body,
        out_shape=jax.ShapeDtypeStruct((_E, _STATE_DIM), jnp.float32),
        interpret=_INTERPRET,
    )(A_states, W_K)

    grid = (_NBLK + 1,)
    idx, w = pl.pallas_call(
        _router_body,
        grid=grid,
        in_specs=[
            pl.BlockSpec((_BLK, _IN_DIM), lambda i: (jnp.minimum(i, _NBLK - 1), 0)),
            pl.BlockSpec((_STATE_DIM, _IN_DIM), lambda i: (0, 0)),
            pl.BlockSpec((_E, _STATE_DIM), lambda i: (0, 0)),
        ],
        out_specs=[
            pl.BlockSpec((_BLK, _KSEL), lambda i: (jnp.maximum(i - 1, 0), 0)),
            pl.BlockSpec((_BLK, _KSEL), lambda i: (jnp.maximum(i - 1, 0), 0)),
        ],
        out_shape=[
            jax.ShapeDtypeStruct((_ROWS, _KSEL), jnp.int32),
            jax.ShapeDtypeStruct((_ROWS, _KSEL), jnp.float32),
        ],
        scratch_shapes=[pltpu.VMEM((2, _BLK, _E), jnp.float32)],
        interpret=_INTERPRET,
    )(h_concat, wq_bf, k_mat)
    return (idx, w, _TAU)


# speedup vs baseline: 1.2132x; 1.2132x over previous
import jax
import jax.numpy as jnp
from jax import lax
from jax.experimental import pallas as pl
from jax.experimental.pallas import tpu as pltpu

_PROGRESS = min(1.0 / 1000.0, 1.0)
_TAU = 2.0 - _PROGRESS * (2.0 - 0.5)

_ROWS = 4096
_IN_DIM = 4096
_STATE_DIM = 2048
_E = 64
_KSEL = 8
_BLK = 512
_NBLK = _ROWS // _BLK
_N_BISECT = 30

_INTERPRET = False


def _keys_body(a_ref, wk_ref, k_ref):
    a = a_ref[...].astype(jnp.bfloat16)
    wk = wk_ref[...].astype(jnp.bfloat16)
    k_ref[...] = lax.dot_general(a, wk, (((1,), (1,)), ((), ())),
                                 preferred_element_type=jnp.float32)


def _entmax_topk(s, idx_ref, w_ref):
    x = s - jnp.max(s, axis=-1, keepdims=True)
    l = jnp.full((_BLK, 1), -2.0, dtype=jnp.float32)
    r = jnp.zeros((_BLK, 1), dtype=jnp.float32)
    for _ in range(_N_BISECT):
        mid = (l + r) * 0.5
        y = jnp.maximum(x - mid, 0.0)
        vm = jnp.sum(y * y, axis=-1, keepdims=True) - 1.0
        gt = vm > 0.0
        l = jnp.where(gt, mid, l)
        r = jnp.where(gt, r, mid)
    tau_b = (l + r) * 0.5
    yy = jnp.maximum(x - tau_b, 0.0)
    sup = yy * yy
    p = sup / (jnp.sum(sup, axis=-1, keepdims=True) + 1e-12)
    iota = lax.broadcasted_iota(jnp.int32, (_BLK, _E), 1)
    vals = []
    idxs = []
    pw = p
    for _ in range(_KSEL):
        m = jnp.max(pw, axis=-1, keepdims=True)
        cand = jnp.where(pw == m, iota, _E)
        am = jnp.min(cand, axis=-1, keepdims=True)
        vals.append(m)
        idxs.append(am)
        pw = jnp.where(iota == am, -1.0, pw)
    v = jnp.concatenate(vals, axis=1)
    w_ref[...] = v / (jnp.sum(v, axis=-1, keepdims=True) + 1e-12)
    idx_ref[...] = jnp.concatenate(idxs, axis=1)


def _router_body(h_ref, wq_ref, k_ref, idx_ref, w_ref, s_scratch):
    i = pl.program_id(0)

    @pl.when(i < _NBLK)
    def _matmul():
        h = h_ref[...].astype(jnp.bfloat16)
        q = lax.dot_general(h, wq_ref[...], (((1,), (1,)), ((), ())),
                            preferred_element_type=jnp.float32)
        qb = q.astype(jnp.bfloat16)
        kb = k_ref[...].astype(jnp.bfloat16)
        s = lax.dot_general(qb, kb, (((1,), (1,)), ((), ())),
                            preferred_element_type=jnp.float32) / _TAU
        s_scratch[lax.rem(i, 2)] = s

    @pl.when(i > 0)
    def _vector():
        _entmax_topk(s_scratch[lax.rem(i + 1, 2)], idx_ref, w_ref)


def kernel(h_concat, A_states, W_Q, W_K):
    wq_bf = W_Q.astype(jnp.bfloat16)
    k_mat = pl.pallas_call(
        _keys_body,
        out_shape=jax.ShapeDtypeStruct((_E, _STATE_DIM), jnp.float32),
        interpret=_INTERPRET,
    )(A_states, W_K)

    grid = (_NBLK + 1,)
    idx, w = pl.pallas_call(
        _router_body,
        grid=grid,
        in_specs=[
            pl.BlockSpec((_BLK, _IN_DIM), lambda i: (jnp.minimum(i, _NBLK - 1), 0)),
            pl.BlockSpec((_STATE_DIM, _IN_DIM), lambda i: (0, 0)),
            pl.BlockSpec((_E, _STATE_DIM), lambda i: (0, 0)),
        ],
        out_specs=[
            pl.BlockSpec((_BLK, _KSEL), lambda i: (jnp.maximum(i - 1, 0), 0)),
            pl.BlockSpec((_BLK, _KSEL), lambda i: (jnp.maximum(i - 1, 0), 0)),
        ],
        out_shape=[
            jax.ShapeDtypeStruct((_ROWS, _KSEL), jnp.int32),
            jax.ShapeDtypeStruct((_ROWS, _KSEL), jnp.float32),
        ],
        scratch_shapes=[pltpu.VMEM((2, _BLK, _E), jnp.float32)],
        interpret=_INTERPRET,
    )(h_concat, wq_bf, k_mat)
    return (idx, w, _TAU)


# speedup vs baseline: 1.3223x; 1.0899x over previous
import jax
import jax.numpy as jnp
from jax import lax
from jax.experimental import pallas as pl
from jax.experimental.pallas import tpu as pltpu

_PROGRESS = min(1.0 / 1000.0, 1.0)
_TAU = 2.0 - _PROGRESS * (2.0 - 0.5)

_ROWS = 4096
_IN_DIM = 4096
_STATE_DIM = 2048
_E = 64
_KSEL = 8
_BLK = 512
_NBLK = _ROWS // _BLK
_N_BISECT = 30

_INTERPRET = False


def _keys_body(a_ref, wk_ref, k_ref):
    a = a_ref[...].astype(jnp.bfloat16)
    wk = wk_ref[...].astype(jnp.bfloat16)
    k_ref[...] = lax.dot_general(a, wk, (((1,), (1,)), ((), ())),
                                 preferred_element_type=jnp.float32)


def _entmax_topk(s, idx_ref, w_ref):
    x = s - jnp.max(s, axis=-1, keepdims=True)
    l = jnp.full((_BLK, 1), -2.0, dtype=jnp.float32)
    r = jnp.zeros((_BLK, 1), dtype=jnp.float32)
    for _ in range(_N_BISECT):
        mid = (l + r) * 0.5
        y = jnp.maximum(x - mid, 0.0)
        vm = jnp.sum(y * y, axis=-1, keepdims=True) - 1.0
        gt = vm > 0.0
        l = jnp.where(gt, mid, l)
        r = jnp.where(gt, r, mid)
    tau_b = (l + r) * 0.5
    yy = jnp.maximum(x - tau_b, 0.0)
    sup = yy * yy
    p = sup / (jnp.sum(sup, axis=-1, keepdims=True) + 1e-12)
    iota = lax.broadcasted_iota(jnp.int32, (_BLK, _E), 1)
    vals = []
    idxs = []
    pw = p
    for _ in range(_KSEL):
        m = jnp.max(pw, axis=-1, keepdims=True)
        cand = jnp.where(pw == m, iota, _E)
        am = jnp.min(cand, axis=-1, keepdims=True)
        vals.append(m)
        idxs.append(am)
        pw = jnp.where(iota == am, -1.0, pw)
    v = jnp.concatenate(vals, axis=1)
    w_ref[...] = v / (jnp.sum(v, axis=-1, keepdims=True) + 1e-12)
    idx_ref[...] = jnp.concatenate(idxs, axis=1)


def _router_body(h_ref, wq_ref, k_ref, idx_ref, w_ref, s_scratch):
    i = pl.program_id(0)

    @pl.when(i < _NBLK)
    def _matmul():
        q = lax.dot_general(h_ref[...], wq_ref[...], (((1,), (1,)), ((), ())),
                            preferred_element_type=jnp.float32)
        s = lax.dot_general(q, k_ref[...], (((1,), (1,)), ((), ())),
                            preferred_element_type=jnp.float32) / _TAU
        s_scratch[lax.rem(i, 2)] = s

    @pl.when(i > 0)
    def _vector():
        _entmax_topk(s_scratch[lax.rem(i + 1, 2)], idx_ref, w_ref)


def kernel(h_concat, A_states, W_Q, W_K):
    k_mat = pl.pallas_call(
        _keys_body,
        out_shape=jax.ShapeDtypeStruct((_E, _STATE_DIM), jnp.float32),
        interpret=_INTERPRET,
    )(A_states, W_K)

    grid = (_NBLK + 1,)
    idx, w = pl.pallas_call(
        _router_body,
        grid=grid,
        in_specs=[
            pl.BlockSpec((_BLK, _IN_DIM), lambda i: (jnp.minimum(i, _NBLK - 1), 0)),
            pl.BlockSpec((_STATE_DIM, _IN_DIM), lambda i: (0, 0)),
            pl.BlockSpec((_E, _STATE_DIM), lambda i: (0, 0)),
        ],
        out_specs=[
            pl.BlockSpec((_BLK, _KSEL), lambda i: (jnp.maximum(i - 1, 0), 0)),
            pl.BlockSpec((_BLK, _KSEL), lambda i: (jnp.maximum(i - 1, 0), 0)),
        ],
        out_shape=[
            jax.ShapeDtypeStruct((_ROWS, _KSEL), jnp.int32),
            jax.ShapeDtypeStruct((_ROWS, _KSEL), jnp.float32),
        ],
        scratch_shapes=[pltpu.VMEM((2, _BLK, _E), jnp.float32)],
        interpret=_INTERPRET,
    )(h_concat, W_Q, k_mat)
    return (idx, w, _TAU)
